# Initial kernel scaffold; baseline (speedup 1.0000x reference)
import jax
import jax.numpy as jnp
from jax.experimental import pallas as pl


def kernel(idx, targets, table):
    raise NotImplementedError("write your pallas kernel here")


# speedup vs baseline: 1.4920x; 1.4920x over previous
import functools

import jax
import jax.numpy as jnp
from jax import lax
from jax.experimental import pallas as pl
from jax.experimental.pallas import tpu as pltpu
from jax.experimental.pallas import tpu_sc as plsc

V = 1000
N = 1024 * 50
NC, NS, L = 2, 16, 16
NW = NC * NS
PW = N // NW
CH = 32
NCH = PW // CH


def _lse_body(table_ref, lse_ref):
    x = table_ref[...]
    m = jnp.max(x, axis=1, keepdims=True)
    s = jnp.sum(jnp.exp(x - m), axis=1, keepdims=True)
    lse_ref[...] = (m + jnp.log(s))[:, 0]


_lse_call = pl.pallas_call(
    _lse_body,
    out_shape=jax.ShapeDtypeStruct((V,), jnp.float32),
)


_mesh = plsc.VectorSubcoreMesh(core_axis_name="c", subcore_axis_name="s")


@functools.partial(
    pl.kernel,
    out_type=[
        jax.ShapeDtypeStruct((N, V), jnp.float32),
        jax.ShapeDtypeStruct((NW, L), jnp.float32),
    ],
    mesh=_mesh,
    scratch_types=[
        pltpu.VMEM((CH,), jnp.int32),
        pltpu.VMEM((CH,), jnp.int32),
        pltpu.VMEM((CH,), jnp.int32),
        pltpu.VMEM((CH,), jnp.float32),
        pltpu.VMEM((CH,), jnp.float32),
        pltpu.VMEM((CH, V), jnp.float32),
        pltpu.VMEM((L,), jnp.float32),
        pltpu.SemaphoreType.DMA,
        pltpu.SemaphoreType.DMA,
    ],
    compiler_params=pltpu.CompilerParams(use_tc_tiling_on_sc=False),
)
def _sc_gather(idx_hbm, tgt_hbm, table_hbm, tabflat_hbm, lse_hbm,
               out_hbm, part_hbm,
               idx_v, tgt_v, fidx_v, tval_v, lseg_v, rows_v, part_v,
               sem, sem2):
    wid = lax.axis_index("s") * NC + lax.axis_index("c")
    base = wid * PW

    def chunk(c, acc):
        off = base + c * CH
        pltpu.sync_copy(idx_hbm.at[pl.ds(off, CH)], idx_v)
        pltpu.sync_copy(tgt_hbm.at[pl.ds(off, CH)], tgt_v)
        row_dma = pltpu.async_copy(table_hbm.at[idx_v], rows_v, sem)
        lse_dma = pltpu.async_copy(lse_hbm.at[idx_v], lseg_v, sem2)
        for g in range(CH // L):
            i16 = idx_v[pl.ds(g * L, L)]
            t16 = tgt_v[pl.ds(g * L, L)]
            fidx_v[pl.ds(g * L, L)] = t16 * V + i16
        lse_dma.wait()
        pltpu.async_copy(tabflat_hbm.at[fidx_v], tval_v, sem2).wait()
        for g in range(CH // L):
            acc = acc + (lseg_v[pl.ds(g * L, L)] - tval_v[pl.ds(g * L, L)])
        row_dma.wait()
        pltpu.sync_copy(rows_v, out_hbm.at[pl.ds(off, CH)])
        return acc

    acc = lax.fori_loop(0, NCH, chunk, jnp.zeros((L,), jnp.float32))
    part_v[...] = acc
    pltpu.sync_copy(part_v, part_hbm.at[wid])


def _loss_body(part_ref, loss_ref):
    loss_ref[0, 0] = jnp.sum(part_ref[...]) * (1.0 / N)


_loss_call = pl.pallas_call(
    _loss_body,
    out_shape=jax.ShapeDtypeStruct((1, 1), jnp.float32),
    out_specs=pl.BlockSpec(memory_space=pltpu.SMEM),
)


@jax.jit
def kernel(idx, targets, table):
    idx_f = idx.reshape(N)
    tgt_f = targets.reshape(N)
    lse = _lse_call(table)
    tabflat = table.T.reshape(V * V)
    logits2, part = _sc_gather(idx_f, tgt_f, table, tabflat, lse)
    loss = _loss_call(part)[0, 0]
    return (logits2, loss)
